# Initial kernel scaffold; baseline (speedup 1.0000x reference)
import functools
import jax
import jax.numpy as jnp
from jax import lax
from jax.experimental import pallas as pl
from jax.experimental.pallas import tpu as pltpu
from jax.experimental.pallas import tpu_sc as plsc

NC, NS = 2, 16
CHUNK = 128


def _mesh():
    return plsc.VectorSubcoreMesh(core_axis_name="c", subcore_axis_name="s")


def _round_up(v, m):
    return (v + m - 1) // m * m


def _prop(g, src2d, dst2d, zrows, N):
    D = g.shape[1]
    EPR = src2d.shape[0]
    CH = EPR // (NC * NS)
    NSP = _round_up(N + 1, NS * CHUNK)
    ZCH = NSP // (NS * CHUNK)
    WB = N // NS

    @functools.partial(
        pl.kernel,
        out_type=jax.ShapeDtypeStruct((NC, N, D), jnp.float32),
        mesh=_mesh(),
        scratch_types=[
            pltpu.VMEM((CH, CHUNK), jnp.int32),
            pltpu.VMEM((CH, CHUNK), jnp.int32),
            pltpu.VMEM((CHUNK, D), jnp.float32),
            pltpu.VMEM_SHARED((NSP, D), jnp.float32),
            pltpu.SemaphoreType.DMA,
        ],
    )
    def k(g_hbm, src_hbm, dst_hbm, z_hbm, out_hbm, idxs, idxd, rows, acc,
          sem):
        c = lax.axis_index("c")
        s = lax.axis_index("s")
        t = c * NS + s
        pltpu.sync_copy(src_hbm.at[pl.ds(t * CH, CH)], idxs)
        pltpu.sync_copy(dst_hbm.at[pl.ds(t * CH, CH)], idxd)
        pltpu.sync_copy(z_hbm, rows)
        for z in range(ZCH):
            pltpu.sync_copy(
                rows, acc.at[pl.ds((s * ZCH + z) * CHUNK, CHUNK)])
        plsc.subcore_barrier()

        def body(j, carry):
            pltpu.async_copy(g_hbm.at[idxs.at[j]], rows, sem).wait()
            pltpu.sync_copy(rows, acc.at[idxd.at[j]], add=True)
            return carry

        lax.fori_loop(0, CH, body, 0)
        plsc.subcore_barrier()
        pltpu.sync_copy(acc.at[pl.ds(s * WB, WB)],
                        out_hbm.at[c].at[pl.ds(s * WB, WB)])

    return k(g, src2d, dst2d, zrows)


def _degree(dst2d, ones_rows, zrows16, N):
    EPR = dst2d.shape[0]
    CH = EPR // (NC * NS)
    NSP = _round_up(N + 1, NS * CHUNK)
    ZCH = NSP // (NS * CHUNK)
    WB = N // NS

    @functools.partial(
        pl.kernel,
        out_type=jax.ShapeDtypeStruct((NC, N, 16), jnp.float32),
        mesh=_mesh(),
        scratch_types=[
            pltpu.VMEM((CH, CHUNK), jnp.int32),
            pltpu.VMEM((CHUNK, 16), jnp.float32),
            pltpu.VMEM_SHARED((NSP, 16), jnp.float32),
        ],
    )
    def k(dst_hbm, ones_hbm, z_hbm, out_hbm, idxd, rows, acc):
        c = lax.axis_index("c")
        s = lax.axis_index("s")
        t = c * NS + s
        pltpu.sync_copy(dst_hbm.at[pl.ds(t * CH, CH)], idxd)
        pltpu.sync_copy(z_hbm, rows)
        for z in range(ZCH):
            pltpu.sync_copy(
                rows, acc.at[pl.ds((s * ZCH + z) * CHUNK, CHUNK)])
        plsc.subcore_barrier()
        pltpu.sync_copy(ones_hbm, rows)

        def body(j, carry):
            pltpu.sync_copy(rows, acc.at[idxd.at[j]], add=True)
            return carry

        lax.fori_loop(0, CH, body, 0)
        plsc.subcore_barrier()
        pltpu.sync_copy(acc.at[pl.ds(s * WB, WB)],
                        out_hbm.at[c].at[pl.ds(s * WB, WB)])

    return k(dst2d, ones_rows, zrows16)


def _dinv_of(cnt0, cnt1):
    return lax.rsqrt(cnt0[:, :1] + cnt1[:, :1] + 1.0)


def _mm_scale(x, W, cnt, bn=500):
    N, Din = x.shape
    Dout = W.shape[1]

    def body(x_ref, w_ref, cnt_ref, o_ref):
        dinv = _dinv_of(cnt_ref[0], cnt_ref[1])
        o_ref[...] = jnp.dot(x_ref[...], w_ref[...],
                             preferred_element_type=jnp.float32) * dinv

    return pl.pallas_call(
        body,
        grid=(N // bn,),
        in_specs=[
            pl.BlockSpec((bn, Din), lambda i: (i, 0)),
            pl.BlockSpec((Din, Dout), lambda i: (0, 0)),
            pl.BlockSpec((NC, bn, 16), lambda i: (0, i, 0)),
        ],
        out_specs=pl.BlockSpec((bn, Dout), lambda i: (i, 0)),
        out_shape=jax.ShapeDtypeStruct((N, Dout), jnp.float32),
    )(x, W, cnt)


def _fuse_mm(s, g, cnt, b, W, bn=500):
    N, D = g.shape
    Dout = W.shape[1]

    def body(s_ref, g_ref, cnt_ref, b_ref, w_ref, o_ref):
        dinv = _dinv_of(cnt_ref[0], cnt_ref[1])
        f = jnp.maximum(
            (s_ref[0] + s_ref[1] + g_ref[...]) * dinv + b_ref[...], 0.0)
        o_ref[...] = jnp.dot(f, w_ref[...],
                             preferred_element_type=jnp.float32) * dinv

    return pl.pallas_call(
        body,
        grid=(N // bn,),
        in_specs=[
            pl.BlockSpec((NC, bn, D), lambda i: (0, i, 0)),
            pl.BlockSpec((bn, D), lambda i: (i, 0)),
            pl.BlockSpec((NC, bn, 16), lambda i: (0, i, 0)),
            pl.BlockSpec((1, D), lambda i: (0, 0)),
            pl.BlockSpec((D, Dout), lambda i: (0, 0)),
        ],
        out_specs=pl.BlockSpec((bn, Dout), lambda i: (i, 0)),
        out_shape=jax.ShapeDtypeStruct((N, Dout), jnp.float32),
    )(s, g, cnt, b, W)


def _epilogue(s, g, cnt, b, bn=500):
    N, D = g.shape

    def body(s_ref, g_ref, cnt_ref, b_ref, o_ref):
        dinv = _dinv_of(cnt_ref[0], cnt_ref[1])
        o_ref[...] = (s_ref[0] + s_ref[1] + g_ref[...]) * dinv + b_ref[...]

    return pl.pallas_call(
        body,
        grid=(N // bn,),
        in_specs=[
            pl.BlockSpec((NC, bn, D), lambda i: (0, i, 0)),
            pl.BlockSpec((bn, D), lambda i: (i, 0)),
            pl.BlockSpec((NC, bn, 16), lambda i: (0, i, 0)),
            pl.BlockSpec((1, D), lambda i: (0, 0)),
        ],
        out_specs=pl.BlockSpec((bn, D), lambda i: (i, 0)),
        out_shape=jax.ShapeDtypeStruct((N, D), jnp.float32),
    )(s, g, cnt, b)


def kernel(x, edge_index, W1, b1, W2, b2, Wmu, bmu, Wls, bls):
    N, _ = x.shape
    E = edge_index.shape[1]
    assert N % NS == 0
    EP = _round_up(E, NC * NS * CHUNK)
    pad = EP - E
    src = jnp.concatenate(
        [edge_index[0], jnp.zeros((pad,), edge_index.dtype)])
    dst = jnp.concatenate(
        [edge_index[1], jnp.full((pad,), N, edge_index.dtype)])
    src2d = src.reshape(EP // CHUNK, CHUNK)
    dst2d = dst.reshape(EP // CHUNK, CHUNK)
    ones16 = jnp.ones((CHUNK, 16), jnp.float32)
    z16 = jnp.zeros((CHUNK, 16), jnp.float32)
    z64 = jnp.zeros((CHUNK, 64), jnp.float32)
    z128 = jnp.zeros((CHUNK, 128), jnp.float32)

    cnt = _degree(dst2d, ones16, z16, N)

    g1 = _mm_scale(x, W1, cnt)
    s1 = _prop(g1, src2d, dst2d, z128, N)

    g2 = _fuse_mm(s1, g1, cnt, b1.reshape(1, -1), W2)
    s2 = _prop(g2, src2d, dst2d, z64, N)

    Wcat = jnp.concatenate([Wmu, Wls], axis=1)
    bcat = jnp.concatenate([bmu, bls]).reshape(1, -1)
    g3 = _fuse_mm(s2, g2, cnt, b2.reshape(1, -1), Wcat)
    s3 = _prop(g3, src2d, dst2d, z128, N)

    out = _epilogue(s3, g3, cnt, bcat)
    return out[:, :64], out[:, 64:]


# speedup vs baseline: 8.1092x; 8.1092x over previous
import functools
import jax
import jax.numpy as jnp
from jax import lax
from jax.experimental import pallas as pl
from jax.experimental.pallas import tpu as pltpu
from jax.experimental.pallas import tpu_sc as plsc

NC, NS = 2, 16
CHUNK = 128


def _mesh():
    return plsc.VectorSubcoreMesh(core_axis_name="c", subcore_axis_name="s")


def _round_up(v, m):
    return (v + m - 1) // m * m


def _prop(g, src2d, dst2d, zrows, N):
    D = g.shape[1]
    EPR = src2d.shape[0]
    CH = EPR // (NC * NS)
    NSP = _round_up(N + 1, NS * CHUNK)
    ZCH = NSP // (NS * CHUNK)
    WBF = NSP // NS
    WBL = N - (NS - 1) * WBF
    assert WBL > 0 and WBF % 8 == 0 and WBL % 8 == 0

    @functools.partial(
        pl.kernel,
        out_type=jax.ShapeDtypeStruct((NC, N, D), jnp.float32),
        mesh=_mesh(),
        scratch_types=[
            pltpu.VMEM((CH, CHUNK), jnp.int32),
            pltpu.VMEM((CH, CHUNK), jnp.int32),
            pltpu.VMEM((CHUNK, D), jnp.float32),
            pltpu.VMEM_SHARED((NSP, D), jnp.float32),
            pltpu.SemaphoreType.DMA,
        ],
    )
    def k(g_hbm, src_hbm, dst_hbm, z_hbm, out_hbm, idxs, idxd, rows, acc,
          sem):
        c = lax.axis_index("c")
        s = lax.axis_index("s")
        t = c * NS + s
        pltpu.sync_copy(src_hbm.at[pl.ds(t * CH, CH)], idxs)
        pltpu.sync_copy(dst_hbm.at[pl.ds(t * CH, CH)], idxd)
        pltpu.sync_copy(z_hbm, rows)
        for z in range(ZCH):
            pltpu.sync_copy(
                rows, acc.at[pl.ds((s * ZCH + z) * CHUNK, CHUNK)])
        plsc.subcore_barrier()

        def body(j, carry):
            pltpu.async_copy(g_hbm.at[idxs.at[j]], rows, sem).wait()
            pltpu.sync_copy(rows, acc.at[idxd.at[j]], add=True)
            return carry

        lax.fori_loop(0, CH, body, 0)
        plsc.subcore_barrier()
        base = s * WBF

        @pl.when(s < NS - 1)
        def _():
            pltpu.sync_copy(acc.at[pl.ds(base, WBF)],
                            out_hbm.at[c].at[pl.ds(base, WBF)])

        @pl.when(s == NS - 1)
        def _():
            pltpu.sync_copy(acc.at[pl.ds(base, WBL)],
                            out_hbm.at[c].at[pl.ds(base, WBL)])

    return k(g, src2d, dst2d, zrows)


def _degree(dst2d, ones_rows, zrows, N):
    EPR = dst2d.shape[0]
    CH = EPR // (NC * NS)
    NSP = _round_up(N + 1, NS * CHUNK)
    ZCH = NSP // (NS * CHUNK)
    WBF = NSP // NS
    WBL = N - (NS - 1) * WBF
    assert WBL > 0 and WBF % 8 == 0 and WBL % 8 == 0

    @functools.partial(
        pl.kernel,
        out_type=jax.ShapeDtypeStruct((NC, N, 128), jnp.float32),
        mesh=_mesh(),
        scratch_types=[
            pltpu.VMEM((CH, CHUNK), jnp.int32),
            pltpu.VMEM((CHUNK, 128), jnp.float32),
            pltpu.VMEM_SHARED((NSP, 128), jnp.float32),
        ],
    )
    def k(dst_hbm, ones_hbm, z_hbm, out_hbm, idxd, rows, acc):
        c = lax.axis_index("c")
        s = lax.axis_index("s")
        t = c * NS + s
        pltpu.sync_copy(dst_hbm.at[pl.ds(t * CH, CH)], idxd)
        pltpu.sync_copy(z_hbm, rows)
        for z in range(ZCH):
            pltpu.sync_copy(
                rows, acc.at[pl.ds((s * ZCH + z) * CHUNK, CHUNK)])
        plsc.subcore_barrier()
        pltpu.sync_copy(ones_hbm, rows)

        def body(j, carry):
            pltpu.sync_copy(rows, acc.at[idxd.at[j]], add=True)
            return carry

        lax.fori_loop(0, CH, body, 0)
        plsc.subcore_barrier()
        base = s * WBF

        @pl.when(s < NS - 1)
        def _():
            pltpu.sync_copy(acc.at[pl.ds(base, WBF)],
                            out_hbm.at[c].at[pl.ds(base, WBF)])

        @pl.when(s == NS - 1)
        def _():
            pltpu.sync_copy(acc.at[pl.ds(base, WBL)],
                            out_hbm.at[c].at[pl.ds(base, WBL)])

    return k(dst2d, ones_rows, zrows)


def _dinv_of(cnt0, cnt1):
    return lax.rsqrt(cnt0[:, :1] + cnt1[:, :1] + 1.0)


def _mm_scale(x, W, cnt, bn=1000):
    N, Din = x.shape
    Dout = W.shape[1]

    def body(x_ref, w_ref, cnt_ref, o_ref):
        dinv = _dinv_of(cnt_ref[0], cnt_ref[1])
        o_ref[...] = jnp.dot(x_ref[...], w_ref[...],
                             preferred_element_type=jnp.float32) * dinv

    return pl.pallas_call(
        body,
        grid=(N // bn,),
        in_specs=[
            pl.BlockSpec((bn, Din), lambda i: (i, 0)),
            pl.BlockSpec((Din, Dout), lambda i: (0, 0)),
            pl.BlockSpec((NC, bn, 128), lambda i: (0, i, 0)),
        ],
        out_specs=pl.BlockSpec((bn, Dout), lambda i: (i, 0)),
        out_shape=jax.ShapeDtypeStruct((N, Dout), jnp.float32),
    )(x, W, cnt)


def _fuse_mm(s, g, cnt, b, W, bn=1000):
    N, D = g.shape
    Dout = W.shape[1]

    def body(s_ref, g_ref, cnt_ref, b_ref, w_ref, o_ref):
        dinv = _dinv_of(cnt_ref[0], cnt_ref[1])
        f = jnp.maximum(
            (s_ref[0] + s_ref[1] + g_ref[...]) * dinv + b_ref[...], 0.0)
        o_ref[...] = jnp.dot(f, w_ref[...],
                             preferred_element_type=jnp.float32) * dinv

    return pl.pallas_call(
        body,
        grid=(N // bn,),
        in_specs=[
            pl.BlockSpec((NC, bn, D), lambda i: (0, i, 0)),
            pl.BlockSpec((bn, D), lambda i: (i, 0)),
            pl.BlockSpec((NC, bn, 128), lambda i: (0, i, 0)),
            pl.BlockSpec((1, D), lambda i: (0, 0)),
            pl.BlockSpec((D, Dout), lambda i: (0, 0)),
        ],
        out_specs=pl.BlockSpec((bn, Dout), lambda i: (i, 0)),
        out_shape=jax.ShapeDtypeStruct((N, Dout), jnp.float32),
    )(s, g, cnt, b, W)


def _epilogue(s, g, cnt, b, bn=1000):
    N, D = g.shape

    def body(s_ref, g_ref, cnt_ref, b_ref, o_ref):
        dinv = _dinv_of(cnt_ref[0], cnt_ref[1])
        o_ref[...] = (s_ref[0] + s_ref[1] + g_ref[...]) * dinv + b_ref[...]

    return pl.pallas_call(
        body,
        grid=(N // bn,),
        in_specs=[
            pl.BlockSpec((NC, bn, D), lambda i: (0, i, 0)),
            pl.BlockSpec((bn, D), lambda i: (i, 0)),
            pl.BlockSpec((NC, bn, 128), lambda i: (0, i, 0)),
            pl.BlockSpec((1, D), lambda i: (0, 0)),
        ],
        out_specs=pl.BlockSpec((bn, D), lambda i: (i, 0)),
        out_shape=jax.ShapeDtypeStruct((N, D), jnp.float32),
    )(s, g, cnt, b)


def kernel(x, edge_index, W1, b1, W2, b2, Wmu, bmu, Wls, bls):
    N, _ = x.shape
    E = edge_index.shape[1]
    assert N % NS == 0
    EP = _round_up(E, NC * NS * CHUNK * 8)
    pad = EP - E
    src = jnp.concatenate(
        [edge_index[0], jnp.zeros((pad,), edge_index.dtype)])
    dst = jnp.concatenate(
        [edge_index[1], jnp.full((pad,), N, edge_index.dtype)])
    src2d = src.reshape(EP // CHUNK, CHUNK)
    dst2d = dst.reshape(EP // CHUNK, CHUNK)
    ones128 = jnp.ones((CHUNK, 128), jnp.float32)
    z128 = jnp.zeros((CHUNK, 128), jnp.float32)

    cnt = _degree(dst2d, ones128, z128, N)

    g1 = _mm_scale(x, W1, cnt)
    s1 = _prop(g1, src2d, dst2d, z128, N)

    h2 = W2.shape[1]
    W2p = jnp.pad(W2, ((0, 0), (0, 128 - h2)))
    b2p = jnp.pad(b2, (0, 128 - h2))
    g2 = _fuse_mm(s1, g1, cnt, b1.reshape(1, -1), W2p)
    s2 = _prop(g2, src2d, dst2d, z128, N)

    Wcat = jnp.concatenate([Wmu, Wls], axis=1)
    Wcatp = jnp.pad(Wcat, ((0, 128 - h2), (0, 0)))
    bcat = jnp.concatenate([bmu, bls]).reshape(1, -1)
    g3 = _fuse_mm(s2, g2, cnt, b2p.reshape(1, -1), Wcatp)
    s3 = _prop(g3, src2d, dst2d, z128, N)

    out = _epilogue(s3, g3, cnt, bcat)
    return out[:, :64], out[:, 64:]


# speedup vs baseline: 8.9630x; 1.1053x over previous
import functools
import jax
import jax.numpy as jnp
from jax import lax
from jax.experimental import pallas as pl
from jax.experimental.pallas import tpu as pltpu
from jax.experimental.pallas import tpu_sc as plsc

NC, NS = 2, 16
CHUNK = 128


def _mesh():
    return plsc.VectorSubcoreMesh(core_axis_name="c", subcore_axis_name="s")


def _round_up(v, m):
    return (v + m - 1) // m * m


def _prop(g, src2d, dst2d, zrows, N):
    D = g.shape[1]
    EPR = src2d.shape[0]
    CH = EPR // (NC * NS)
    NSP = _round_up(N + 1, NS * CHUNK)
    ZCH = NSP // (NS * CHUNK)
    WBF = NSP // NS
    WBL = N - (NS - 1) * WBF
    assert WBL > 0 and WBF % 8 == 0 and WBL % 8 == 0

    IB = 16
    assert CH % IB == 0 and IB % 2 == 0
    NBLK = CH // IB

    @functools.partial(
        pl.kernel,
        out_type=jax.ShapeDtypeStruct((NC, N, D), jnp.float32),
        mesh=_mesh(),
        scratch_types=[
            pltpu.VMEM((IB, CHUNK), jnp.int32),
            pltpu.VMEM((IB, CHUNK), jnp.int32),
            pltpu.VMEM((CHUNK, D), jnp.float32),
            pltpu.VMEM((CHUNK, D), jnp.float32),
            pltpu.VMEM_SHARED((NSP, D), jnp.float32),
            pltpu.SemaphoreType.DMA,
            pltpu.SemaphoreType.DMA,
        ],
    )
    def k(g_hbm, src_hbm, dst_hbm, z_hbm, out_hbm, idxs, idxd, rows0,
          rows1, acc, sem0, sem1):
        c = lax.axis_index("c")
        s = lax.axis_index("s")
        t = c * NS + s
        pltpu.sync_copy(z_hbm, rows0)
        for z in range(ZCH):
            pltpu.sync_copy(
                rows0, acc.at[pl.ds((s * ZCH + z) * CHUNK, CHUNK)])
        plsc.subcore_barrier()

        def blk(bi, carry):
            pltpu.sync_copy(src_hbm.at[pl.ds(t * CH + bi * IB, IB)], idxs)
            pltpu.sync_copy(dst_hbm.at[pl.ds(t * CH + bi * IB, IB)], idxd)
            cp0 = pltpu.async_copy(g_hbm.at[idxs.at[0]], rows0, sem0)
            for u in range(IB // 2):
                j0, j1 = 2 * u, 2 * u + 1
                cp1 = pltpu.async_copy(g_hbm.at[idxs.at[j1]], rows1, sem1)
                cp0.wait()
                pltpu.sync_copy(rows0, acc.at[idxd.at[j0]], add=True)
                if j1 + 1 < IB:
                    cp0 = pltpu.async_copy(
                        g_hbm.at[idxs.at[j1 + 1]], rows0, sem0)
                cp1.wait()
                pltpu.sync_copy(rows1, acc.at[idxd.at[j1]], add=True)
            return carry

        lax.fori_loop(0, NBLK, blk, 0)
        plsc.subcore_barrier()
        base = s * WBF

        @pl.when(s < NS - 1)
        def _():
            pltpu.sync_copy(acc.at[pl.ds(base, WBF)],
                            out_hbm.at[c].at[pl.ds(base, WBF)])

        @pl.when(s == NS - 1)
        def _():
            pltpu.sync_copy(acc.at[pl.ds(base, WBL)],
                            out_hbm.at[c].at[pl.ds(base, WBL)])

    return k(g, src2d, dst2d, zrows)


def _degree(dst2d, ones_rows, zrows, N):
    EPR = dst2d.shape[0]
    CH = EPR // (NC * NS)
    NSP = _round_up(N + 1, NS * CHUNK)
    ZCH = NSP // (NS * CHUNK)
    WBF = NSP // NS
    WBL = N - (NS - 1) * WBF
    assert WBL > 0 and WBF % 8 == 0 and WBL % 8 == 0

    @functools.partial(
        pl.kernel,
        out_type=jax.ShapeDtypeStruct((NC, N, 128), jnp.float32),
        mesh=_mesh(),
        scratch_types=[
            pltpu.VMEM((CH, CHUNK), jnp.int32),
            pltpu.VMEM((CHUNK, 128), jnp.float32),
            pltpu.VMEM_SHARED((NSP, 128), jnp.float32),
        ],
    )
    def k(dst_hbm, ones_hbm, z_hbm, out_hbm, idxd, rows, acc):
        c = lax.axis_index("c")
        s = lax.axis_index("s")
        t = c * NS + s
        pltpu.sync_copy(dst_hbm.at[pl.ds(t * CH, CH)], idxd)
        pltpu.sync_copy(z_hbm, rows)
        for z in range(ZCH):
            pltpu.sync_copy(
                rows, acc.at[pl.ds((s * ZCH + z) * CHUNK, CHUNK)])
        plsc.subcore_barrier()
        pltpu.sync_copy(ones_hbm, rows)

        def body(j, carry):
            pltpu.sync_copy(rows, acc.at[idxd.at[j]], add=True)
            return carry

        lax.fori_loop(0, CH, body, 0)
        plsc.subcore_barrier()
        base = s * WBF

        @pl.when(s < NS - 1)
        def _():
            pltpu.sync_copy(acc.at[pl.ds(base, WBF)],
                            out_hbm.at[c].at[pl.ds(base, WBF)])

        @pl.when(s == NS - 1)
        def _():
            pltpu.sync_copy(acc.at[pl.ds(base, WBL)],
                            out_hbm.at[c].at[pl.ds(base, WBL)])

    return k(dst2d, ones_rows, zrows)


def _dinv_of(cnt0, cnt1):
    return lax.rsqrt(cnt0[:, :1] + cnt1[:, :1] + 1.0)


def _mm_scale(x, W, cnt, bn=1000):
    N, Din = x.shape
    Dout = W.shape[1]

    def body(x_ref, w_ref, cnt_ref, o_ref):
        dinv = _dinv_of(cnt_ref[0], cnt_ref[1])
        o_ref[...] = jnp.dot(x_ref[...], w_ref[...],
                             preferred_element_type=jnp.float32) * dinv

    return pl.pallas_call(
        body,
        grid=(N // bn,),
        in_specs=[
            pl.BlockSpec((bn, Din), lambda i: (i, 0)),
            pl.BlockSpec((Din, Dout), lambda i: (0, 0)),
            pl.BlockSpec((NC, bn, 128), lambda i: (0, i, 0)),
        ],
        out_specs=pl.BlockSpec((bn, Dout), lambda i: (i, 0)),
        out_shape=jax.ShapeDtypeStruct((N, Dout), jnp.float32),
    )(x, W, cnt)


def _fuse_mm(s, g, cnt, b, W, bn=1000):
    N, D = g.shape
    Dout = W.shape[1]

    def body(s_ref, g_ref, cnt_ref, b_ref, w_ref, o_ref):
        dinv = _dinv_of(cnt_ref[0], cnt_ref[1])
        f = jnp.maximum(
            (s_ref[0] + s_ref[1] + g_ref[...]) * dinv + b_ref[...], 0.0)
        o_ref[...] = jnp.dot(f, w_ref[...],
                             preferred_element_type=jnp.float32) * dinv

    return pl.pallas_call(
        body,
        grid=(N // bn,),
        in_specs=[
            pl.BlockSpec((NC, bn, D), lambda i: (0, i, 0)),
            pl.BlockSpec((bn, D), lambda i: (i, 0)),
            pl.BlockSpec((NC, bn, 128), lambda i: (0, i, 0)),
            pl.BlockSpec((1, D), lambda i: (0, 0)),
            pl.BlockSpec((D, Dout), lambda i: (0, 0)),
        ],
        out_specs=pl.BlockSpec((bn, Dout), lambda i: (i, 0)),
        out_shape=jax.ShapeDtypeStruct((N, Dout), jnp.float32),
    )(s, g, cnt, b, W)


def _epilogue(s, g, cnt, b, bn=1000):
    N, D = g.shape

    def body(s_ref, g_ref, cnt_ref, b_ref, o_ref):
        dinv = _dinv_of(cnt_ref[0], cnt_ref[1])
        o_ref[...] = (s_ref[0] + s_ref[1] + g_ref[...]) * dinv + b_ref[...]

    return pl.pallas_call(
        body,
        grid=(N // bn,),
        in_specs=[
            pl.BlockSpec((NC, bn, D), lambda i: (0, i, 0)),
            pl.BlockSpec((bn, D), lambda i: (i, 0)),
            pl.BlockSpec((NC, bn, 128), lambda i: (0, i, 0)),
            pl.BlockSpec((1, D), lambda i: (0, 0)),
        ],
        out_specs=pl.BlockSpec((bn, D), lambda i: (i, 0)),
        out_shape=jax.ShapeDtypeStruct((N, D), jnp.float32),
    )(s, g, cnt, b)


def kernel(x, edge_index, W1, b1, W2, b2, Wmu, bmu, Wls, bls):
    N, _ = x.shape
    E = edge_index.shape[1]
    assert N % NS == 0
    EP = _round_up(E, NC * NS * CHUNK * 8)
    pad = EP - E
    src = jnp.concatenate(
        [edge_index[0], jnp.zeros((pad,), edge_index.dtype)])
    dst = jnp.concatenate(
        [edge_index[1], jnp.full((pad,), N, edge_index.dtype)])
    src2d = src.reshape(EP // CHUNK, CHUNK)
    dst2d = dst.reshape(EP // CHUNK, CHUNK)
    ones128 = jnp.ones((CHUNK, 128), jnp.float32)
    z128 = jnp.zeros((CHUNK, 128), jnp.float32)

    cnt = _degree(dst2d, ones128, z128, N)

    g1 = _mm_scale(x, W1, cnt)
    s1 = _prop(g1, src2d, dst2d, z128, N)

    h2 = W2.shape[1]
    W2p = jnp.pad(W2, ((0, 0), (0, 128 - h2)))
    b2p = jnp.pad(b2, (0, 128 - h2))
    g2 = _fuse_mm(s1, g1, cnt, b1.reshape(1, -1), W2p)
    s2 = _prop(g2, src2d, dst2d, z128, N)

    Wcat = jnp.concatenate([Wmu, Wls], axis=1)
    Wcatp = jnp.pad(Wcat, ((0, 128 - h2), (0, 0)))
    bcat = jnp.concatenate([bmu, bls]).reshape(1, -1)
    g3 = _fuse_mm(s2, g2, cnt, b2p.reshape(1, -1), Wcatp)
    s3 = _prop(g3, src2d, dst2d, z128, N)

    out = _epilogue(s3, g3, cnt, bcat)
    return out[:, :64], out[:, 64:]
